# baseline (speedup 1.0000x reference)
import functools

import jax
import jax.numpy as jnp
from jax import lax
from jax.experimental import pallas as pl
from jax.experimental.pallas import tpu as pltpu
from jax.experimental.pallas import tpu_sc as plsc

B = 4096
F = 128
NB = 20
NC = 2
NS = 16
NW = NC * NS
BPW = B // NW
TB = 2048
NT = B // TB


CH = 4
CIDX = CH * NB
NCH = BPW // CH
RING = 4


def _sc_gather_body(idx_all, w_user_gmf, w_item_gmf,
                    w_user_sage, w_item_sage, uid_out, item_out, usage_out,
                    isage_out, idq_u, idq_i, idxf, idxf2, b0, b1, b2, b3, acc,
                    idb, s0, s1, s2, s3, semw, semx):
    wid = lax.axis_index("s") * NC + lax.axis_index("c")
    base = wid * BPW
    bufs = (b0, b1, b2, b3)
    sems = (s0, s1, s2, s3)

    pltpu.sync_copy(idx_all.at[pl.ds(base, BPW)], idq_u)
    pltpu.sync_copy(idx_all.at[pl.ds(B + base, BPW)], idq_i)
    cp_idu = pltpu.async_copy(w_user_gmf.at[idq_u], idb.at[pl.ds(0, BPW)], semw)
    cp_idi = pltpu.async_copy(w_item_gmf.at[idq_i], idb.at[pl.ds(BPW, BPW)], semw)

    def fire(table, ixf, c, q):
        return pltpu.async_copy(
            table.at[ixf.at[pl.ds(c * CIDX, CIDX)]], bufs[q], sems[q])

    pltpu.sync_copy(idx_all.at[pl.ds(2 * B + base * NB, BPW * NB)], idxf)
    cp_x2 = pltpu.async_copy(
        idx_all.at[pl.ds(2 * B + B * NB + base * NB, BPW * NB)], idxf2, semx)
    for q in range(RING):
        fire(w_item_sage, idxf, q, q)
    cp_idu.wait()
    cp_idi.wait()
    cp_wu = pltpu.async_copy(idb.at[pl.ds(0, BPW)],
                             uid_out.at[pl.ds(base, BPW)], semw)
    cp_wi = pltpu.async_copy(idb.at[pl.ds(BPW, BPW)],
                             item_out.at[pl.ds(base, BPW)], semw)
    cp_x2.wait()

    def jbody(j, carry):
        for q in range(RING):
            c = RING * j + q
            pltpu.make_async_copy(
                w_item_sage.at[idxf.at[pl.ds(0, CIDX)]], bufs[q],
                sems[q]).wait()
            b = bufs[q]
            arow0 = CH * c - jnp.where(c >= NCH, CH * NCH, 0)

            def brbody(br, carry, b=b, arow0=arow0):
                row0 = NB * br
                cs = tuple(b[row0, pl.ds(16 * v, 16)]
                           + b[row0 + 1, pl.ds(16 * v, 16)]
                           for v in range(F // 16))

                def nbody(m, cs, b=b, row0=row0):
                    r = row0 + 2 * m
                    return tuple(cs[v] + b[r, pl.ds(16 * v, 16)]
                                 + b[r + 1, pl.ds(16 * v, 16)]
                                 for v in range(F // 16))

                cs = lax.fori_loop(1, NB // 2, nbody, cs)
                for v in range(F // 16):
                    acc[arow0 + br, pl.ds(16 * v, 16)] = cs[v]
                return carry

            lax.fori_loop(0, CH, brbody, 0)

            cn = c + RING

            @pl.when(cn < NCH)
            def _():
                fire(w_item_sage, idxf, cn, q)

            @pl.when((cn >= NCH) & (cn < 2 * NCH))
            def _():
                fire(w_user_sage, idxf2, cn - NCH, q)

            @pl.when(c == NCH - 1)
            def _():
                pltpu.sync_copy(acc, usage_out.at[pl.ds(base, BPW)])
        return carry

    lax.fori_loop(0, 2 * NCH // RING, jbody, 0)
    pltpu.sync_copy(acc, isage_out.at[pl.ds(base, BPW)])
    cp_wu.wait()
    cp_wi.wait()


@functools.lru_cache(maxsize=1)
def _sc_gather():
    return pl.kernel(
        _sc_gather_body,
        out_type=(
            jax.ShapeDtypeStruct((B, F), jnp.float32),
            jax.ShapeDtypeStruct((B, F), jnp.float32),
            jax.ShapeDtypeStruct((B, F), jnp.float32),
            jax.ShapeDtypeStruct((B, F), jnp.float32),
        ),
        mesh=plsc.VectorSubcoreMesh(core_axis_name="c", subcore_axis_name="s"),
        scratch_types=[
            pltpu.VMEM((BPW,), jnp.int32),
            pltpu.VMEM((BPW,), jnp.int32),
            pltpu.VMEM((BPW * NB,), jnp.int32),
            pltpu.VMEM((BPW * NB,), jnp.int32),
            pltpu.VMEM((CIDX, F), jnp.float32),
            pltpu.VMEM((CIDX, F), jnp.float32),
            pltpu.VMEM((CIDX, F), jnp.float32),
            pltpu.VMEM((CIDX, F), jnp.float32),
            pltpu.VMEM((BPW, F), jnp.float32),
            pltpu.VMEM((2 * BPW, F), jnp.float32),
            pltpu.SemaphoreType.DMA,
            pltpu.SemaphoreType.DMA,
            pltpu.SemaphoreType.DMA,
            pltpu.SemaphoreType.DMA,
            pltpu.SemaphoreType.DMA,
            pltpu.SemaphoreType.DMA,
        ],
    )


def _tc_mlp_body(uid_ref, item_ref, us_ref, is_ref, g_ref, a_ref, o_ref,
                 wall_ref, wg_ref, wa_ref, wo_ref, wcu_ref, wci_ref, ball_ref,
                 bcu_ref, bci_ref, wp1_ref, wp2_ref, bp1_ref, bp2_ref,
                 out_ref):
    f32 = jnp.float32
    a1 = wall_ref[0:F, :]
    tsmall = jnp.concatenate([
        jnp.dot(wo_ref[...], wall_ref[F:2 * F, :], preferred_element_type=f32),
        jnp.dot(wa_ref[...], wall_ref[2 * F:3 * F, :],
                preferred_element_type=f32),
        jnp.dot(wg_ref[...], wall_ref[3 * F:4 * F, :],
                preferred_element_type=f32),
        jnp.zeros((2, F), f32),
    ], axis=0)
    g = g_ref[0]
    a = a_ref[0]
    o = o_ref[0]
    cols = lax.broadcasted_iota(jnp.int32, (TB, 32), 1)
    sh = ((cols == o) | (cols == a + 21) | (cols == g + 28)).astype(f32)
    c1 = wcu_ref[0:F, :]
    c2 = wcu_ref[F:2 * F, :] * (1.0 / NB)
    e1 = jnp.dot(a1, c1, preferred_element_type=f32)
    tsc = jnp.dot(tsmall, c1, preferred_element_type=f32)
    bu = jnp.dot(ball_ref[...], c1, preferred_element_type=f32) + bcu_ref[...]
    uf = (jnp.dot(uid_ref[...], e1, preferred_element_type=f32)
          + jnp.dot(sh, tsc, preferred_element_type=f32)
          + jnp.dot(us_ref[...], c2, preferred_element_type=f32)
          + bu)
    d1 = wci_ref[0:F, :]
    d2 = wci_ref[F:2 * F, :] * (1.0 / NB)
    itf = (jnp.dot(item_ref[...], d1, preferred_element_type=f32)
           + jnp.dot(is_ref[...], d2, preferred_element_type=f32)
           + bci_ref[...])
    pvec = jnp.dot(wp1_ref[...], wp2_ref[...],
                   preferred_element_type=f32)
    cconst = jnp.sum(bp1_ref[...] * wp2_ref[...]) + bp2_ref[0, 0]
    out_ref[...] = jnp.dot(uf * itf, pvec, preferred_element_type=f32) + cconst


def _tc_mlp(uid_e, item_e, usage_s, isage_s, g3, a3, o3, w_all, w_g, w_a,
            w_o, w_cu, w_ci, b_all, b_cu, b_ci, wp1, wp2, bp1, bp2):
    emb_spec = pl.BlockSpec((TB, F), lambda i: (i, 0))
    idx_spec = pl.BlockSpec((1, TB, 1), lambda i: (i, 0, 0))

    def full(x):
        r = len(x.shape)
        return pl.BlockSpec(x.shape, lambda i, _r=r: (0,) * _r)

    return pl.pallas_call(
        _tc_mlp_body,
        grid=(NT,),
        in_specs=[emb_spec, emb_spec, emb_spec, emb_spec,
                  idx_spec, idx_spec, idx_spec,
                  full(w_all), full(w_g), full(w_a), full(w_o),
                  full(w_cu), full(w_ci),
                  full(b_all), full(b_cu), full(b_ci),
                  full(wp1), full(wp2), full(bp1), full(bp2)],
        out_specs=pl.BlockSpec((TB, 1), lambda i: (i, 0)),
        out_shape=jax.ShapeDtypeStruct((B, 1), jnp.float32),
    )(uid_e, item_e, usage_s, isage_s, g3, a3, o3, w_all, w_g, w_a, w_o,
      w_cu, w_ci, b_all, b_cu, b_ci, wp1, wp2, bp1, bp2)


def kernel(user, item, user_gender, user_age, user_occupation, user_neighbor,
           item_neighbor, W_user_gmf, W_item_gmf, W_user_sage, W_item_sage,
           W_gender, W_age, W_occ, W_all, b_all, W_cu, b_cu, W_ci, b_ci,
           W_p1, b_p1, W_p2, b_p2):
    i32 = jnp.int32
    if user.dtype != i32:
        user = user.astype(i32)
    if item.dtype != i32:
        item = item.astype(i32)
    if user_neighbor.dtype != i32:
        user_neighbor = user_neighbor.astype(i32)
    if item_neighbor.dtype != i32:
        item_neighbor = item_neighbor.astype(i32)
    un_flat = user_neighbor.reshape(-1)
    in_flat = item_neighbor.reshape(-1)

    idx_all = jnp.concatenate([user, item, un_flat, in_flat])
    uid_e, item_e, usage_s, isage_s = _sc_gather()(
        idx_all, W_user_gmf, W_item_gmf, W_user_sage, W_item_sage)

    g3 = user_gender.reshape(NT, TB, 1)
    a3 = user_age.reshape(NT, TB, 1)
    o3 = user_occupation.reshape(NT, TB, 1)

    pred = _tc_mlp(uid_e, item_e, usage_s, isage_s, g3, a3, o3,
                   W_all, W_gender, W_age, W_occ, W_cu, W_ci,
                   b_all.reshape(1, F), b_cu.reshape(1, F), b_ci.reshape(1, F),
                   W_p1, W_p2, b_p1.reshape(8, 1), b_p2.reshape(1, 1))
    return pred.reshape(-1)


# speedup vs baseline: 1.0357x; 1.0357x over previous
import functools

import jax
import jax.numpy as jnp
from jax import lax
from jax.experimental import pallas as pl
from jax.experimental.pallas import tpu as pltpu
from jax.experimental.pallas import tpu_sc as plsc

B = 4096
F = 128
NB = 20
NC = 2
NS = 16
NW = NC * NS
BPW = B // NW
TB = 2048
NT = B // TB


CH = 4
CIDX = CH * NB
NCH = BPW // CH
RING = 4


def _sc_gather_body(idx_all, w_user_gmf, w_item_gmf,
                    w_user_sage, w_item_sage, uid_out, item_out, usage_out,
                    isage_out, idq_u, idq_i, idxf, idxf2, b0, b1, b2, b3, acc,
                    idb, s0, s1, s2, s3, semw, semx):
    wid = lax.axis_index("s") * NC + lax.axis_index("c")
    base = wid * BPW
    bufs = (b0, b1, b2, b3)
    sems = (s0, s1, s2, s3)

    pltpu.sync_copy(idx_all.at[pl.ds(base, BPW)], idq_u)
    pltpu.sync_copy(idx_all.at[pl.ds(B + base, BPW)], idq_i)
    cp_idu = pltpu.async_copy(w_user_gmf.at[idq_u], idb.at[pl.ds(0, BPW)], semw)
    cp_idi = pltpu.async_copy(w_item_gmf.at[idq_i], idb.at[pl.ds(BPW, BPW)], semw)

    def fire(table, ixf, c, q):
        return pltpu.async_copy(
            table.at[ixf.at[pl.ds(c * CIDX, CIDX)]], bufs[q], sems[q])

    pltpu.sync_copy(idx_all.at[pl.ds(2 * B + base * NB, BPW * NB)], idxf)
    cp_x2 = pltpu.async_copy(
        idx_all.at[pl.ds(2 * B + B * NB + base * NB, BPW * NB)], idxf2, semx)
    for q in range(RING):
        fire(w_item_sage, idxf, q, q)
    cp_idu.wait()
    cp_idi.wait()
    cp_wu = pltpu.async_copy(idb.at[pl.ds(0, BPW)],
                             uid_out.at[pl.ds(base, BPW)], semw)
    cp_wi = pltpu.async_copy(idb.at[pl.ds(BPW, BPW)],
                             item_out.at[pl.ds(base, BPW)], semw)
    cp_x2.wait()

    def jbody(j, carry):
        for q in range(RING):
            c = RING * j + q
            pltpu.make_async_copy(
                w_item_sage.at[idxf.at[pl.ds(0, CIDX)]], bufs[q],
                sems[q]).wait()
            b = bufs[q]
            arow0 = CH * c - jnp.where(c >= NCH, CH * NCH, 0)

            def brbody(br, carry, b=b, arow0=arow0):
                row0 = NB * br
                cs = tuple(b[row0, pl.ds(16 * v, 16)]
                           + b[row0 + 1, pl.ds(16 * v, 16)]
                           for v in range(F // 16))

                def nbody(m, cs, b=b, row0=row0):
                    r = row0 + 2 * m
                    return tuple(cs[v] + b[r, pl.ds(16 * v, 16)]
                                 + b[r + 1, pl.ds(16 * v, 16)]
                                 for v in range(F // 16))

                cs = lax.fori_loop(1, NB // 2, nbody, cs)
                for v in range(F // 16):
                    acc[arow0 + br, pl.ds(16 * v, 16)] = cs[v]
                return carry

            lax.fori_loop(0, CH, brbody, 0)

            cn = c + RING

            @pl.when(cn < NCH)
            def _():
                fire(w_item_sage, idxf, cn, q)

            @pl.when((cn >= NCH) & (cn < 2 * NCH))
            def _():
                fire(w_user_sage, idxf2, cn - NCH, q)

            @pl.when(c == NCH - 1)
            def _():
                pltpu.sync_copy(acc, usage_out.at[pl.ds(base, BPW)])
        return carry

    lax.fori_loop(0, 2 * NCH // RING, jbody, 0)
    pltpu.sync_copy(acc, isage_out.at[pl.ds(base, BPW)])
    cp_wu.wait()
    cp_wi.wait()


@functools.lru_cache(maxsize=1)
def _sc_gather():
    return pl.kernel(
        _sc_gather_body,
        out_type=(
            jax.ShapeDtypeStruct((B, F), jnp.float32),
            jax.ShapeDtypeStruct((B, F), jnp.float32),
            jax.ShapeDtypeStruct((B, F), jnp.float32),
            jax.ShapeDtypeStruct((B, F), jnp.float32),
        ),
        mesh=plsc.VectorSubcoreMesh(core_axis_name="c", subcore_axis_name="s"),
        scratch_types=[
            pltpu.VMEM((BPW,), jnp.int32),
            pltpu.VMEM((BPW,), jnp.int32),
            pltpu.VMEM((BPW * NB,), jnp.int32),
            pltpu.VMEM((BPW * NB,), jnp.int32),
            pltpu.VMEM((CIDX, F), jnp.float32),
            pltpu.VMEM((CIDX, F), jnp.float32),
            pltpu.VMEM((CIDX, F), jnp.float32),
            pltpu.VMEM((CIDX, F), jnp.float32),
            pltpu.VMEM((BPW, F), jnp.float32),
            pltpu.VMEM((2 * BPW, F), jnp.float32),
            pltpu.SemaphoreType.DMA,
            pltpu.SemaphoreType.DMA,
            pltpu.SemaphoreType.DMA,
            pltpu.SemaphoreType.DMA,
            pltpu.SemaphoreType.DMA,
            pltpu.SemaphoreType.DMA,
        ],
    )


def _tc_mlp_body(uid_ref, item_ref, us_ref, is_ref, g_ref, a_ref, o_ref,
                 wall_ref, wg_ref, wa_ref, wo_ref, wcu_ref, wci_ref, ball_ref,
                 bcu_ref, bci_ref, wp1_ref, wp2_ref, bp1_ref, bp2_ref,
                 out_ref):
    f32 = jnp.float32
    a1 = wall_ref[0:F, :]
    tsmall = jnp.concatenate([
        jnp.dot(wo_ref[...], wall_ref[F:2 * F, :], preferred_element_type=f32),
        jnp.dot(wa_ref[...], wall_ref[2 * F:3 * F, :],
                preferred_element_type=f32),
        jnp.dot(wg_ref[...], wall_ref[3 * F:4 * F, :],
                preferred_element_type=f32),
        jnp.zeros((2, F), f32),
    ], axis=0)
    g = jnp.reshape(g_ref[0], (TB, 1))
    a = jnp.reshape(a_ref[0], (TB, 1))
    o = jnp.reshape(o_ref[0], (TB, 1))
    cols = lax.broadcasted_iota(jnp.int32, (TB, 32), 1)
    sh = ((cols == o) | (cols == a + 21) | (cols == g + 28)).astype(f32)
    c1 = wcu_ref[0:F, :]
    c2 = wcu_ref[F:2 * F, :] * (1.0 / NB)
    e1 = jnp.dot(a1, c1, preferred_element_type=f32)
    tsc = jnp.dot(tsmall, c1, preferred_element_type=f32)
    bu = jnp.dot(ball_ref[...], c1, preferred_element_type=f32) + bcu_ref[...]
    uf = (jnp.dot(uid_ref[...], e1, preferred_element_type=f32)
          + jnp.dot(sh, tsc, preferred_element_type=f32)
          + jnp.dot(us_ref[...], c2, preferred_element_type=f32)
          + bu)
    d1 = wci_ref[0:F, :]
    d2 = wci_ref[F:2 * F, :] * (1.0 / NB)
    itf = (jnp.dot(item_ref[...], d1, preferred_element_type=f32)
           + jnp.dot(is_ref[...], d2, preferred_element_type=f32)
           + bci_ref[...])
    pvec = jnp.dot(wp1_ref[...], wp2_ref[...],
                   preferred_element_type=f32)
    cconst = jnp.sum(bp1_ref[...] * wp2_ref[...]) + bp2_ref[0, 0]
    out_ref[...] = jnp.dot(uf * itf, pvec, preferred_element_type=f32) + cconst


def _tc_mlp(uid_e, item_e, usage_s, isage_s, g3, a3, o3, w_all, w_g, w_a,
            w_o, w_cu, w_ci, b_all, b_cu, b_ci, wp1, wp2, bp1, bp2):
    emb_spec = pl.BlockSpec((TB, F), lambda i: (i, 0))
    idx_spec = pl.BlockSpec((1, 1, TB), lambda i: (i, 0, 0))

    def full(x):
        r = len(x.shape)
        return pl.BlockSpec(x.shape, lambda i, _r=r: (0,) * _r)

    return pl.pallas_call(
        _tc_mlp_body,
        grid=(NT,),
        in_specs=[emb_spec, emb_spec, emb_spec, emb_spec,
                  idx_spec, idx_spec, idx_spec,
                  full(w_all), full(w_g), full(w_a), full(w_o),
                  full(w_cu), full(w_ci),
                  full(b_all), full(b_cu), full(b_ci),
                  full(wp1), full(wp2), full(bp1), full(bp2)],
        out_specs=pl.BlockSpec((TB, 1), lambda i: (i, 0)),
        out_shape=jax.ShapeDtypeStruct((B, 1), jnp.float32),
    )(uid_e, item_e, usage_s, isage_s, g3, a3, o3, w_all, w_g, w_a, w_o,
      w_cu, w_ci, b_all, b_cu, b_ci, wp1, wp2, bp1, bp2)


def kernel(user, item, user_gender, user_age, user_occupation, user_neighbor,
           item_neighbor, W_user_gmf, W_item_gmf, W_user_sage, W_item_sage,
           W_gender, W_age, W_occ, W_all, b_all, W_cu, b_cu, W_ci, b_ci,
           W_p1, b_p1, W_p2, b_p2):
    i32 = jnp.int32
    if user.dtype != i32:
        user = user.astype(i32)
    if item.dtype != i32:
        item = item.astype(i32)
    if user_neighbor.dtype != i32:
        user_neighbor = user_neighbor.astype(i32)
    if item_neighbor.dtype != i32:
        item_neighbor = item_neighbor.astype(i32)
    un_flat = user_neighbor.reshape(-1)
    in_flat = item_neighbor.reshape(-1)

    idx_all = jnp.concatenate([user, item, un_flat, in_flat])
    uid_e, item_e, usage_s, isage_s = _sc_gather()(
        idx_all, W_user_gmf, W_item_gmf, W_user_sage, W_item_sage)

    g3 = user_gender.reshape(NT, 1, TB)
    a3 = user_age.reshape(NT, 1, TB)
    o3 = user_occupation.reshape(NT, 1, TB)

    pred = _tc_mlp(uid_e, item_e, usage_s, isage_s, g3, a3, o3,
                   W_all, W_gender, W_age, W_occ, W_cu, W_ci,
                   b_all.reshape(1, F), b_cu.reshape(1, F), b_ci.reshape(1, F),
                   W_p1, W_p2, b_p1.reshape(8, 1), b_p2.reshape(1, 1))
    return pred.reshape(-1)


# speedup vs baseline: 1.0667x; 1.0299x over previous
import functools

import jax
import jax.numpy as jnp
from jax import lax
from jax.experimental import pallas as pl
from jax.experimental.pallas import tpu as pltpu
from jax.experimental.pallas import tpu_sc as plsc

B = 4096
F = 128
NB = 20
NC = 2
NS = 16
NW = NC * NS
BPW = B // NW
TB = 2048
NT = B // TB


CH = 4
CIDX = CH * NB
NCH = BPW // CH
RING = 4


def _sc_gather_body(idx_all, w_user_gmf, w_item_gmf,
                    w_user_sage, w_item_sage, uid_out, item_out, usage_out,
                    isage_out, idq_u, idq_i, idxf, idxf2, b0, b1, b2, b3, acc,
                    idb, s0, s1, s2, s3, semw, semx):
    wid = lax.axis_index("s") * NC + lax.axis_index("c")
    base = wid * BPW
    bufs = (b0, b1, b2, b3)
    sems = (s0, s1, s2, s3)

    pltpu.sync_copy(idx_all.at[pl.ds(base, BPW)], idq_u)
    pltpu.sync_copy(idx_all.at[pl.ds(B + base, BPW)], idq_i)
    cp_idu = pltpu.async_copy(w_user_gmf.at[idq_u], idb.at[pl.ds(0, BPW)], semw)
    cp_idi = pltpu.async_copy(w_item_gmf.at[idq_i], idb.at[pl.ds(BPW, BPW)], semw)

    def fire(table, ixf, c, q):
        return pltpu.async_copy(
            table.at[ixf.at[pl.ds(c * CIDX, CIDX)]], bufs[q], sems[q])

    pltpu.sync_copy(idx_all.at[pl.ds(2 * B + base * NB, BPW * NB)], idxf)
    cp_x2 = pltpu.async_copy(
        idx_all.at[pl.ds(2 * B + B * NB + base * NB, BPW * NB)], idxf2, semx)
    for q in range(RING):
        fire(w_item_sage, idxf, q, q)
    cp_idu.wait()
    cp_idi.wait()
    cp_wu = pltpu.async_copy(idb.at[pl.ds(0, BPW)],
                             uid_out.at[pl.ds(base, BPW)], semw)
    cp_wi = pltpu.async_copy(idb.at[pl.ds(BPW, BPW)],
                             item_out.at[pl.ds(base, BPW)], semw)
    cp_x2.wait()

    def jbody(j, carry):
        for q in range(RING):
            c = RING * j + q
            pltpu.make_async_copy(
                w_item_sage.at[idxf.at[pl.ds(0, CIDX)]], bufs[q],
                sems[q]).wait()
            b = bufs[q]
            arow0 = CH * c - jnp.where(c >= NCH, CH * NCH, 0)

            def brbody(br, carry, b=b, arow0=arow0):
                row0 = NB * br
                cs = tuple(b[row0, pl.ds(16 * v, 16)]
                           + b[row0 + 1, pl.ds(16 * v, 16)]
                           for v in range(F // 16))

                def nbody(m, cs, b=b, row0=row0):
                    r = row0 + 2 * m
                    return tuple(cs[v] + b[r, pl.ds(16 * v, 16)]
                                 + b[r + 1, pl.ds(16 * v, 16)]
                                 for v in range(F // 16))

                cs = lax.fori_loop(1, NB // 2, nbody, cs)
                for v in range(F // 16):
                    acc[arow0 + br, pl.ds(16 * v, 16)] = cs[v]
                return carry

            lax.fori_loop(0, CH, brbody, 0)

            cn = c + RING

            @pl.when(cn < NCH)
            def _():
                fire(w_item_sage, idxf, cn, q)

            @pl.when((cn >= NCH) & (cn < 2 * NCH))
            def _():
                fire(w_user_sage, idxf2, cn - NCH, q)

            @pl.when(c == NCH - 1)
            def _():
                pltpu.sync_copy(acc, usage_out.at[pl.ds(base, BPW)])
        return carry

    lax.fori_loop(0, 2 * NCH // RING, jbody, 0)
    pltpu.sync_copy(acc, isage_out.at[pl.ds(base, BPW)])
    cp_wu.wait()
    cp_wi.wait()


@functools.lru_cache(maxsize=1)
def _sc_gather():
    return pl.kernel(
        _sc_gather_body,
        out_type=(
            jax.ShapeDtypeStruct((B, F), jnp.float32),
            jax.ShapeDtypeStruct((B, F), jnp.float32),
            jax.ShapeDtypeStruct((B, F), jnp.float32),
            jax.ShapeDtypeStruct((B, F), jnp.float32),
        ),
        mesh=plsc.VectorSubcoreMesh(core_axis_name="c", subcore_axis_name="s"),
        scratch_types=[
            pltpu.VMEM((BPW,), jnp.int32),
            pltpu.VMEM((BPW,), jnp.int32),
            pltpu.VMEM((BPW * NB,), jnp.int32),
            pltpu.VMEM((BPW * NB,), jnp.int32),
            pltpu.VMEM((CIDX, F), jnp.float32),
            pltpu.VMEM((CIDX, F), jnp.float32),
            pltpu.VMEM((CIDX, F), jnp.float32),
            pltpu.VMEM((CIDX, F), jnp.float32),
            pltpu.VMEM((BPW, F), jnp.float32),
            pltpu.VMEM((2 * BPW, F), jnp.float32),
            pltpu.SemaphoreType.DMA,
            pltpu.SemaphoreType.DMA,
            pltpu.SemaphoreType.DMA,
            pltpu.SemaphoreType.DMA,
            pltpu.SemaphoreType.DMA,
            pltpu.SemaphoreType.DMA,
        ],
    )


def _tc_mlp_body(uid_ref, item_ref, us_ref, is_ref, g_ref, a_ref, o_ref,
                 wall_ref, wg_ref, wa_ref, wo_ref, wcu_ref, wci_ref, ball_ref,
                 bcu_ref, bci_ref, wp1t_ref, wp2_ref, bp1_ref, bp2_ref,
                 out_ref):
    f32 = jnp.float32
    a1 = wall_ref[0:F, :]
    tsmall = jnp.concatenate([
        jnp.dot(wo_ref[...], wall_ref[F:2 * F, :], preferred_element_type=f32),
        jnp.dot(wa_ref[...], wall_ref[2 * F:3 * F, :],
                preferred_element_type=f32),
        jnp.dot(wg_ref[...], wall_ref[3 * F:4 * F, :],
                preferred_element_type=f32),
        jnp.zeros((2, F), f32),
    ], axis=0)
    g = jnp.reshape(g_ref[0], (TB, 1))
    a = jnp.reshape(a_ref[0], (TB, 1))
    o = jnp.reshape(o_ref[0], (TB, 1))
    cols = lax.broadcasted_iota(jnp.int32, (TB, 32), 1)
    sh = ((cols == o) | (cols == a + 21) | (cols == g + 28)).astype(f32)
    c1 = wcu_ref[0:F, :]
    c2 = wcu_ref[F:2 * F, :] * (1.0 / NB)
    e1 = jnp.dot(a1, c1, preferred_element_type=f32)
    tsc = jnp.dot(tsmall, c1, preferred_element_type=f32)
    bu = jnp.dot(ball_ref[...], c1, preferred_element_type=f32) + bcu_ref[...]
    uf = (jnp.dot(uid_ref[...], e1, preferred_element_type=f32)
          + jnp.dot(sh, tsc, preferred_element_type=f32)
          + jnp.dot(us_ref[...], c2, preferred_element_type=f32)
          + bu)
    d1 = wci_ref[0:F, :]
    d2 = wci_ref[F:2 * F, :] * (1.0 / NB)
    itf = (jnp.dot(item_ref[...], d1, preferred_element_type=f32)
           + jnp.dot(is_ref[...], d2, preferred_element_type=f32)
           + bci_ref[...])
    pvec = jnp.sum(wp1t_ref[...] * wp2_ref[...], axis=0)
    cconst = jnp.sum(bp1_ref[...] * wp2_ref[...]) + bp2_ref[0, 0]
    out_ref[...] = jnp.sum(uf * itf * pvec, axis=1) + cconst


def _tc_mlp(uid_e, item_e, usage_s, isage_s, g3, a3, o3, w_all, w_g, w_a,
            w_o, w_cu, w_ci, b_all, b_cu, b_ci, wp1t, wp2, bp1, bp2):
    emb_spec = pl.BlockSpec((TB, F), lambda i: (i, 0))
    idx_spec = pl.BlockSpec((1, 1, TB), lambda i: (i, 0, 0))

    def full(x):
        r = len(x.shape)
        return pl.BlockSpec(x.shape, lambda i, _r=r: (0,) * _r)

    return pl.pallas_call(
        _tc_mlp_body,
        grid=(NT,),
        in_specs=[emb_spec, emb_spec, emb_spec, emb_spec,
                  idx_spec, idx_spec, idx_spec,
                  full(w_all), full(w_g), full(w_a), full(w_o),
                  full(w_cu), full(w_ci),
                  full(b_all), full(b_cu), full(b_ci),
                  full(wp1t), full(wp2), full(bp1), full(bp2)],
        out_specs=pl.BlockSpec((TB,), lambda i: (i,)),
        out_shape=jax.ShapeDtypeStruct((B,), jnp.float32),
    )(uid_e, item_e, usage_s, isage_s, g3, a3, o3, w_all, w_g, w_a, w_o,
      w_cu, w_ci, b_all, b_cu, b_ci, wp1t, wp2, bp1, bp2)


def kernel(user, item, user_gender, user_age, user_occupation, user_neighbor,
           item_neighbor, W_user_gmf, W_item_gmf, W_user_sage, W_item_sage,
           W_gender, W_age, W_occ, W_all, b_all, W_cu, b_cu, W_ci, b_ci,
           W_p1, b_p1, W_p2, b_p2):
    i32 = jnp.int32
    if user.dtype != i32:
        user = user.astype(i32)
    if item.dtype != i32:
        item = item.astype(i32)
    if user_neighbor.dtype != i32:
        user_neighbor = user_neighbor.astype(i32)
    if item_neighbor.dtype != i32:
        item_neighbor = item_neighbor.astype(i32)
    un_flat = user_neighbor.reshape(-1)
    in_flat = item_neighbor.reshape(-1)

    idx_all = jnp.concatenate([user, item, un_flat, in_flat])
    uid_e, item_e, usage_s, isage_s = _sc_gather()(
        idx_all, W_user_gmf, W_item_gmf, W_user_sage, W_item_sage)

    g3 = user_gender.reshape(NT, 1, TB)
    a3 = user_age.reshape(NT, 1, TB)
    o3 = user_occupation.reshape(NT, 1, TB)

    pred = _tc_mlp(uid_e, item_e, usage_s, isage_s, g3, a3, o3,
                   W_all, W_gender, W_age, W_occ, W_cu, W_ci,
                   b_all.reshape(1, F), b_cu.reshape(1, F), b_ci.reshape(1, F),
                   W_p1.T, W_p2, b_p1.reshape(8, 1), b_p2.reshape(1, 1))
    return pred


# speedup vs baseline: 1.0703x; 1.0034x over previous
import functools

import jax
import jax.numpy as jnp
from jax import lax
from jax.experimental import pallas as pl
from jax.experimental.pallas import tpu as pltpu
from jax.experimental.pallas import tpu_sc as plsc

B = 4096
F = 128
NB = 20
NC = 2
NS = 16
NW = NC * NS
BPW = B // NW
TB = 2048
NT = B // TB


CH = 4
CIDX = CH * NB
NCH = BPW // CH
RING = 4


def _sc_gather_body(idx_all, w_user_gmf, w_item_gmf,
                    w_user_sage, w_item_sage, uid_out, item_out, usage_out,
                    isage_out, idq_u, idq_i, idxf, idxf2, b0, b1, b2, b3, acc,
                    idb, s0, s1, s2, s3, semw, semx):
    wid = lax.axis_index("s") * NC + lax.axis_index("c")
    base = wid * BPW
    bufs = (b0, b1, b2, b3)
    sems = (s0, s1, s2, s3)

    pltpu.sync_copy(idx_all.at[pl.ds(base, BPW)], idq_u)
    pltpu.sync_copy(idx_all.at[pl.ds(B + base, BPW)], idq_i)
    cp_idu = pltpu.async_copy(w_user_gmf.at[idq_u], idb.at[pl.ds(0, BPW)], semw)
    cp_idi = pltpu.async_copy(w_item_gmf.at[idq_i], idb.at[pl.ds(BPW, BPW)], semw)

    def fire(table, ixf, c, q):
        return pltpu.async_copy(
            table.at[ixf.at[pl.ds(c * CIDX, CIDX)]], bufs[q], sems[q])

    pltpu.sync_copy(idx_all.at[pl.ds(2 * B + base * NB, BPW * NB)], idxf)
    cp_x2 = pltpu.async_copy(
        idx_all.at[pl.ds(2 * B + B * NB + base * NB, BPW * NB)], idxf2, semx)
    for q in range(RING):
        fire(w_item_sage, idxf, q, q)
    cp_idu.wait()
    cp_idi.wait()
    cp_wu = pltpu.async_copy(idb.at[pl.ds(0, BPW)],
                             uid_out.at[pl.ds(base, BPW)], semw)
    cp_wi = pltpu.async_copy(idb.at[pl.ds(BPW, BPW)],
                             item_out.at[pl.ds(base, BPW)], semw)
    cp_x2.wait()

    def jbody(j, carry):
        for q in range(RING):
            c = RING * j + q
            pltpu.make_async_copy(
                w_item_sage.at[idxf.at[pl.ds(0, CIDX)]], bufs[q],
                sems[q]).wait()
            b = bufs[q]
            arow0 = CH * c - jnp.where(c >= NCH, CH * NCH, 0)

            def brbody(br, carry, b=b, arow0=arow0):
                row0 = NB * br
                cs = tuple(b[row0, pl.ds(16 * v, 16)]
                           + b[row0 + 1, pl.ds(16 * v, 16)]
                           for v in range(F // 16))

                def nbody(m, cs, b=b, row0=row0):
                    r = row0 + 2 * m
                    return tuple(cs[v] + b[r, pl.ds(16 * v, 16)]
                                 + b[r + 1, pl.ds(16 * v, 16)]
                                 for v in range(F // 16))

                cs = lax.fori_loop(1, NB // 2, nbody, cs)
                for v in range(F // 16):
                    acc[arow0 + br, pl.ds(16 * v, 16)] = cs[v]
                return carry

            lax.fori_loop(0, CH, brbody, 0)

            cn = c + RING

            @pl.when(cn < NCH)
            def _():
                fire(w_item_sage, idxf, cn, q)

            @pl.when((cn >= NCH) & (cn < 2 * NCH))
            def _():
                fire(w_user_sage, idxf2, cn - NCH, q)

            @pl.when(c == NCH - 1)
            def _():
                pltpu.sync_copy(acc, usage_out.at[pl.ds(base, BPW)])
        return carry

    lax.fori_loop(0, 2 * NCH // RING, jbody, 0)
    pltpu.sync_copy(acc, isage_out.at[pl.ds(base, BPW)])
    cp_wu.wait()
    cp_wi.wait()


@functools.lru_cache(maxsize=1)
def _sc_gather():
    return pl.kernel(
        _sc_gather_body,
        out_type=(
            jax.ShapeDtypeStruct((B, F), jnp.float32),
            jax.ShapeDtypeStruct((B, F), jnp.float32),
            jax.ShapeDtypeStruct((B, F), jnp.float32),
            jax.ShapeDtypeStruct((B, F), jnp.float32),
        ),
        mesh=plsc.VectorSubcoreMesh(core_axis_name="c", subcore_axis_name="s"),
        scratch_types=[
            pltpu.VMEM((BPW,), jnp.int32),
            pltpu.VMEM((BPW,), jnp.int32),
            pltpu.VMEM((BPW * NB,), jnp.int32),
            pltpu.VMEM((BPW * NB,), jnp.int32),
            pltpu.VMEM((CIDX, F), jnp.float32),
            pltpu.VMEM((CIDX, F), jnp.float32),
            pltpu.VMEM((CIDX, F), jnp.float32),
            pltpu.VMEM((CIDX, F), jnp.float32),
            pltpu.VMEM((BPW, F), jnp.float32),
            pltpu.VMEM((2 * BPW, F), jnp.float32),
            pltpu.SemaphoreType.DMA,
            pltpu.SemaphoreType.DMA,
            pltpu.SemaphoreType.DMA,
            pltpu.SemaphoreType.DMA,
            pltpu.SemaphoreType.DMA,
            pltpu.SemaphoreType.DMA,
        ],
    )


def _tc_mlp_body(uid_ref, item_ref, us_ref, is_ref, g_ref, a_ref, o_ref,
                 wall_ref, wg_ref, wa_ref, wo_ref, wcu_ref, wci_ref, ball_ref,
                 bcu_ref, bci_ref, wp1t_ref, wp2_ref, bp1_ref, bp2_ref,
                 out_ref):
    f32 = jnp.float32
    a1 = wall_ref[0:F, :]
    tsmall = jnp.concatenate([
        jnp.dot(wo_ref[...], wall_ref[F:2 * F, :], preferred_element_type=f32),
        jnp.dot(wa_ref[...], wall_ref[2 * F:3 * F, :],
                preferred_element_type=f32),
        jnp.dot(wg_ref[...], wall_ref[3 * F:4 * F, :],
                preferred_element_type=f32),
        jnp.zeros((2, F), f32),
    ], axis=0)
    g = jnp.reshape(g_ref[0], (TB, 1))
    a = jnp.reshape(a_ref[0], (TB, 1))
    o = jnp.reshape(o_ref[0], (TB, 1))
    cols = lax.broadcasted_iota(jnp.int32, (TB, 32), 1)
    sh = ((cols == o) | (cols == a + 21) | (cols == g + 28)).astype(f32)
    c1 = wcu_ref[0:F, :]
    c2 = wcu_ref[F:2 * F, :] * (1.0 / NB)
    e1 = jnp.dot(a1, c1, preferred_element_type=f32)
    tsc = jnp.dot(tsmall, c1, preferred_element_type=f32)
    bu = jnp.dot(ball_ref[...], c1, preferred_element_type=f32) + bcu_ref[...]
    uf = (jnp.dot(uid_ref[...], e1, preferred_element_type=f32)
          + jnp.dot(sh, tsc, preferred_element_type=f32)
          + jnp.dot(us_ref[...], c2, preferred_element_type=f32)
          + bu)
    d1 = wci_ref[0:F, :]
    d2 = wci_ref[F:2 * F, :] * (1.0 / NB)
    itf = (jnp.dot(item_ref[...], d1, preferred_element_type=f32)
           + jnp.dot(is_ref[...], d2, preferred_element_type=f32)
           + bci_ref[...])
    pvec = jnp.sum(wp1t_ref[...] * wp2_ref[...], axis=0)
    cconst = jnp.sum(bp1_ref[...] * wp2_ref[...]) + bp2_ref[0, 0]
    out_ref[...] = jnp.sum(uf * itf * pvec, axis=1) + cconst


def _tc_mlp(uid_e, item_e, usage_s, isage_s, g3, a3, o3, w_all, w_g, w_a,
            w_o, w_cu, w_ci, b_all, b_cu, b_ci, wp1t, wp2, bp1, bp2):
    emb_spec = pl.BlockSpec((TB, F), lambda i: (i, 0))
    idx_spec = pl.BlockSpec((1, 1, TB), lambda i: (i, 0, 0))

    def full(x):
        r = len(x.shape)
        return pl.BlockSpec(x.shape, lambda i, _r=r: (0,) * _r)

    return pl.pallas_call(
        _tc_mlp_body,
        grid=(NT,),
        in_specs=[emb_spec, emb_spec, emb_spec, emb_spec,
                  idx_spec, idx_spec, idx_spec,
                  full(w_all), full(w_g), full(w_a), full(w_o),
                  full(w_cu), full(w_ci),
                  full(b_all), full(b_cu), full(b_ci),
                  full(wp1t), full(wp2), full(bp1), full(bp2)],
        out_specs=pl.BlockSpec((TB,), lambda i: (i,)),
        out_shape=jax.ShapeDtypeStruct((B,), jnp.float32),
    )(uid_e, item_e, usage_s, isage_s, g3, a3, o3, w_all, w_g, w_a, w_o,
      w_cu, w_ci, b_all, b_cu, b_ci, wp1t, wp2, bp1, bp2)


def kernel(user, item, user_gender, user_age, user_occupation, user_neighbor,
           item_neighbor, W_user_gmf, W_item_gmf, W_user_sage, W_item_sage,
           W_gender, W_age, W_occ, W_all, b_all, W_cu, b_cu, W_ci, b_ci,
           W_p1, b_p1, W_p2, b_p2):
    i32 = jnp.int32
    if user.dtype != i32:
        user = user.astype(i32)
    if item.dtype != i32:
        item = item.astype(i32)
    if user_neighbor.dtype != i32:
        user_neighbor = user_neighbor.astype(i32)
    if item_neighbor.dtype != i32:
        item_neighbor = item_neighbor.astype(i32)
    un_flat = user_neighbor.reshape(-1)
    in_flat = item_neighbor.reshape(-1)

    idx_all = jnp.concatenate([user, item, un_flat, in_flat])
    uid_e, item_e, usage_s, isage_s = _sc_gather()(
        idx_all, W_user_gmf, W_item_gmf, W_user_sage, W_item_sage)

    if user_gender.dtype != i32:
        user_gender = user_gender.astype(i32)
    if user_age.dtype != i32:
        user_age = user_age.astype(i32)
    if user_occupation.dtype != i32:
        user_occupation = user_occupation.astype(i32)
    g3 = user_gender.reshape(NT, 1, TB)
    a3 = user_age.reshape(NT, 1, TB)
    o3 = user_occupation.reshape(NT, 1, TB)

    pred = _tc_mlp(uid_e, item_e, usage_s, isage_s, g3, a3, o3,
                   W_all, W_gender, W_age, W_occ, W_cu, W_ci,
                   b_all.reshape(1, F), b_cu.reshape(1, F), b_ci.reshape(1, F),
                   W_p1.T, W_p2, b_p1.reshape(8, 1), b_p2.reshape(1, 1))
    return pred
